# baseline (speedup 1.0000x reference)
import functools

import numpy as np
import jax
import jax.numpy as jnp
from jax import lax
from jax.experimental import pallas as pl
from jax.experimental.pallas import tpu as pltpu
from jax.experimental.pallas import tpu_sc as plsc


def _pos_encoding_np(max_len, d_model):
    pos = np.expand_dims(np.arange(0, max_len), axis=1).astype(np.float64)
    index = np.expand_dims(np.arange(0, d_model), axis=0).astype(np.float64)
    pe = pos / np.power(10000, (index - index % 2) / np.float32(d_model))
    pe[:, 0::2] = np.sin(pe[:, 0::2])
    pe[:, 1::2] = np.cos(pe[:, 1::2])
    return pe.astype(np.float32)


def _sc_info():
    try:
        info = plsc.get_sparse_core_info()
        return info.num_cores, info.num_subcores
    except Exception:
        return 2, 16


@functools.lru_cache(maxsize=None)
def _build(B, S, V, D, idx_dtype):
    NC, NS = _sc_info()
    NW = NC * NS
    N = B * S
    assert N % NW == 0
    R = N // NW
    C = 32
    assert R % C == 0
    NCHUNK = R // C
    assert D % 16 == 0
    KV = D // 16
    assert S % R == 0 or R % S == 0
    scale = float(np.sqrt(np.float32(D)))

    mesh = plsc.VectorSubcoreMesh(core_axis_name="c", subcore_axis_name="s")

    @functools.partial(
        pl.kernel,
        out_type=jax.ShapeDtypeStruct((N, D), jnp.float32),
        mesh=mesh,
        scratch_types=[
            pltpu.VMEM((R,), jnp.int32),
            pltpu.VMEM((C, D), jnp.float32),
            pltpu.VMEM((C, D), jnp.float32),
            pltpu.SemaphoreType.DMA,
        ],
    )
    def emb_kernel(idx_hbm, table_hbm, pe_hbm, out_hbm, idx_v, buf, pebuf, sem):
        wid = lax.axis_index("s") * NC + lax.axis_index("c")
        base = wid * R
        pltpu.sync_copy(idx_hbm.at[pl.ds(base, R)], idx_v)
        pos_base = lax.rem(base, S)
        for c in range(NCHUNK):
            gat = pltpu.async_copy(
                table_hbm.at[idx_v.at[pl.ds(c * C, C)]], buf, sem)
            pltpu.sync_copy(pe_hbm.at[pl.ds(pos_base + c * C, C)], pebuf)
            gat.wait()

            @plsc.parallel_loop(0, C)
            def _rows(r):
                @plsc.parallel_loop(0, KV, unroll=8)
                def _cols(k):
                    buf[r, pl.ds(k * 16, 16)] = (
                        buf[r, pl.ds(k * 16, 16)] * scale
                        + pebuf[r, pl.ds(k * 16, 16)])

            pltpu.sync_copy(buf, out_hbm.at[pl.ds(base + c * C, C)])

    return emb_kernel


def kernel(sequences, table):
    B, S = sequences.shape
    V, D = table.shape
    idx = sequences.reshape(B * S).astype(jnp.int32)
    pe = jnp.asarray(_pos_encoding_np(S, D))
    emb_kernel = _build(B, S, V, D, str(sequences.dtype))
    out = emb_kernel(idx, table, pe)
    return out.reshape(B, S, D)


# speedup vs baseline: 1.3211x; 1.3211x over previous
import functools

import numpy as np
import jax
import jax.numpy as jnp
from jax import lax
from jax.experimental import pallas as pl
from jax.experimental.pallas import tpu as pltpu
from jax.experimental.pallas import tpu_sc as plsc


def _pos_encoding_np(max_len, d_model):
    pos = np.expand_dims(np.arange(0, max_len), axis=1).astype(np.float64)
    index = np.expand_dims(np.arange(0, d_model), axis=0).astype(np.float64)
    pe = pos / np.power(10000, (index - index % 2) / np.float32(d_model))
    pe[:, 0::2] = np.sin(pe[:, 0::2])
    pe[:, 1::2] = np.cos(pe[:, 1::2])
    return pe.astype(np.float32)


def _sc_info():
    try:
        info = plsc.get_sparse_core_info()
        return info.num_cores, info.num_subcores
    except Exception:
        return 2, 16


@functools.lru_cache(maxsize=None)
def _build(B, S, V, D):
    NC, NS = _sc_info()
    NW = NC * NS
    assert S % NW == 0
    P = S // NW
    C = 32
    assert P % C == 0
    PH = P // C
    NCHUNK = PH * B
    assert D % 16 == 0
    KV = D // 16
    scale = float(np.sqrt(np.float32(D)))

    mesh = plsc.VectorSubcoreMesh(core_axis_name="c", subcore_axis_name="s")

    @functools.partial(
        pl.kernel,
        out_type=jax.ShapeDtypeStruct((B * S, D), jnp.float32),
        mesh=mesh,
        scratch_types=[
            pltpu.VMEM((B, P), jnp.int32),
            pltpu.VMEM((2, C, D), jnp.float32),
            pltpu.VMEM((C, D), jnp.float32),
            pltpu.SemaphoreType.DMA,
            pltpu.SemaphoreType.DMA,
            pltpu.SemaphoreType.DMA,
            pltpu.SemaphoreType.DMA,
        ],
    )
    def emb_kernel(seq_hbm, table_hbm, pe_hbm, out_hbm,
                   idx_v, buf, pebuf, g0, g1, s0, s1):
        wid = lax.axis_index("s") * NC + lax.axis_index("c")
        wpos = wid * P
        gsem = (g0, g1)
        ssem = (s0, s1)

        for b in range(B):
            pltpu.sync_copy(seq_hbm.at[b, pl.ds(wpos, P)], idx_v.at[b])
        pltpu.sync_copy(pe_hbm.at[pl.ds(wpos, C)], pebuf)

        def issue_gather(j):
            ph, b = divmod(j, B)
            p = j % 2
            return pltpu.async_copy(
                table_hbm.at[idx_v.at[b, pl.ds(ph * C, C)]],
                buf.at[p], gsem[p])

        def compute_and_store(j, gat):
            ph, b = divmod(j, B)
            p = j % 2
            gat.wait()

            @plsc.parallel_loop(0, C)
            def _rows(r):
                @plsc.parallel_loop(0, KV, unroll=8)
                def _cols(k):
                    buf[p, r, pl.ds(k * 16, 16)] = (
                        buf[p, r, pl.ds(k * 16, 16)] * scale
                        + pebuf[r, pl.ds(k * 16, 16)])

            return pltpu.async_copy(
                buf.at[p], out_hbm.at[pl.ds(b * S + wpos + ph * C, C)],
                ssem[p])

        gats = {0: issue_gather(0)}
        stores = {}
        for j in range(1, NCHUNK + 1):
            if j < NCHUNK:
                if j - 2 in stores:
                    stores.pop(j - 2).wait()
                gats[j] = issue_gather(j)
            jj = j - 1
            if jj > 0 and jj % B == 0:
                pltpu.sync_copy(pe_hbm.at[pl.ds(wpos + (jj // B) * C, C)],
                                pebuf)
            stores[jj] = compute_and_store(jj, gats.pop(jj))
        for st in stores.values():
            st.wait()

    return emb_kernel


def kernel(sequences, table):
    B, S = sequences.shape
    V, D = table.shape
    pe = jnp.asarray(_pos_encoding_np(S, D))
    emb_kernel = _build(B, S, V, D)
    out = emb_kernel(sequences.astype(jnp.int32), table, pe)
    return out.reshape(B, S, D)
